# baseline (speedup 1.0000x reference)
import functools

import jax
import jax.numpy as jnp
from jax import lax
from jax.experimental import pallas as pl
from jax.experimental.pallas import tpu as pltpu
from jax.experimental.pallas import tpu_sc as plsc

B = 16384
K = 26
VOCAB = 100
DIM = 128

NC = 2
NS = 16
NW = NC * NS
RPW = B // NW
LANES = 16
GROUPS = RPW // LANES

_mesh = plsc.VectorSubcoreMesh(
    core_axis_name="c", subcore_axis_name="s", num_cores=NC, num_subcores=NS
)


@functools.partial(
    pl.kernel,
    out_type=jax.ShapeDtypeStruct((B, VOCAB), jnp.float32),
    mesh=_mesh,
    scratch_types=[
        pltpu.VMEM((RPW, K), jnp.int32),
        pltpu.VMEM((RPW, K), jnp.float32),
        pltpu.VMEM((RPW, VOCAB), jnp.float32),
    ],
    compiler_params=pltpu.CompilerParams(
        use_tc_tiling_on_sc=False, needs_layout_passes=False
    ),
)
def _hist_kernel(idx_hbm, vals_hbm, h_hbm, idx_v, vals_v, h_v):
    wid = lax.axis_index("s") * NC + lax.axis_index("c")
    row0 = wid * RPW
    pltpu.sync_copy(idx_hbm.at[pl.ds(row0, RPW), :], idx_v)
    pltpu.sync_copy(vals_hbm.at[pl.ds(row0, RPW), :], vals_v)

    zeros16 = jnp.zeros((LANES,), jnp.float32)
    _ZCOLS = (0, 16, 32, 48, 64, 80, 84)

    def zero_body(r, carry):
        for c in _ZCOLS:
            h_v[r, pl.ds(c, LANES)] = zeros16
        return carry

    lax.fori_loop(0, RPW, zero_body, 0)

    lane = lax.iota(jnp.int32, LANES)

    def scatter_body(g, carry):
        rows = g * LANES + lane
        for k in range(K):
            ks = jnp.full((LANES,), k, jnp.int32)
            iv = plsc.load_gather(idx_v, [rows, ks])
            vv = plsc.load_gather(vals_v, [rows, ks])
            plsc.addupdate_scatter(h_v, [rows, iv], vv)
        return carry

    lax.fori_loop(0, GROUPS, scatter_body, 0)

    pltpu.sync_copy(h_v, h_hbm.at[pl.ds(row0, RPW), :])


_BM = 1024


def _mm_body(h_ref, w_ref, o_ref):
    o_ref[:] = jnp.dot(h_ref[:], w_ref[:], preferred_element_type=jnp.float32)


_matmul = pl.pallas_call(
    _mm_body,
    grid=(B // _BM,),
    in_specs=[
        pl.BlockSpec((_BM, VOCAB), lambda i: (i, 0)),
        pl.BlockSpec((VOCAB, DIM), lambda i: (0, 0)),
    ],
    out_specs=pl.BlockSpec((_BM, DIM), lambda i: (i, 0)),
    out_shape=jax.ShapeDtypeStruct((B, DIM), jnp.float32),
)


def kernel(idx, vals, weight):
    h = _hist_kernel(idx.astype(jnp.int32), vals)
    return _matmul(h, weight)


# speedup vs baseline: 1.5158x; 1.5158x over previous
import functools

import jax
import jax.numpy as jnp
from jax import lax
from jax.experimental import pallas as pl
from jax.experimental.pallas import tpu as pltpu
from jax.experimental.pallas import tpu_sc as plsc

B = 16384
K = 26
VOCAB = 100
VPAD = 128
DIM = 128

NC = 2
NS = 16
NW = NC * NS
RPW = B // NW
LANES = 16
GROUPS = RPW // LANES

_mesh = plsc.VectorSubcoreMesh(
    core_axis_name="c", subcore_axis_name="s", num_cores=NC, num_subcores=NS
)


@functools.partial(
    pl.kernel,
    out_type=jax.ShapeDtypeStruct((B, VPAD), jnp.float32),
    mesh=_mesh,
    scratch_types=[
        pltpu.VMEM((RPW * K,), jnp.int32),
        pltpu.VMEM((RPW * K,), jnp.float32),
        pltpu.VMEM((RPW, VPAD), jnp.float32),
    ],
    compiler_params=pltpu.CompilerParams(
        use_tc_tiling_on_sc=False, needs_layout_passes=False
    ),
)
def _hist_kernel(idx_hbm, vals_hbm, h_hbm, idx_v, vals_v, h_v):
    wid = lax.axis_index("s") * NC + lax.axis_index("c")
    row0 = wid * RPW
    ebase = wid * (RPW * K)
    pltpu.sync_copy(idx_hbm.at[pl.ds(ebase, RPW * K)], idx_v)
    pltpu.sync_copy(vals_hbm.at[pl.ds(ebase, RPW * K)], vals_v)

    zeros16 = jnp.zeros((LANES,), jnp.float32)

    def zero_body(r, carry):
        for c in range(0, VPAD, LANES):
            h_v[r, pl.ds(c, LANES)] = zeros16
        return carry

    lax.fori_loop(0, RPW, zero_body, 0)

    lane = lax.iota(jnp.int32, LANES)
    lane_k = lane * K

    def scatter_body(g, carry):
        rows = g * LANES + lane
        eoffs = g * (LANES * K) + lane_k
        for k in range(K):
            iv = plsc.load_gather(idx_v, [eoffs + k])
            vv = plsc.load_gather(vals_v, [eoffs + k])
            plsc.addupdate_scatter(h_v, [rows, iv], vv)
        return carry

    lax.fori_loop(0, GROUPS, scatter_body, 0)

    pltpu.sync_copy(h_v, h_hbm.at[pl.ds(row0, RPW), :])


_BM = 2048


def _mm_body(h_ref, w_ref, o_ref):
    w = jnp.concatenate(
        [w_ref[:], jnp.zeros((VPAD - VOCAB, DIM), jnp.float32)], axis=0
    )
    o_ref[:] = jnp.dot(h_ref[:], w, preferred_element_type=jnp.float32)


_matmul = pl.pallas_call(
    _mm_body,
    grid=(B // _BM,),
    in_specs=[
        pl.BlockSpec((_BM, VPAD), lambda i: (i, 0)),
        pl.BlockSpec((VOCAB, DIM), lambda i: (0, 0)),
    ],
    out_specs=pl.BlockSpec((_BM, DIM), lambda i: (i, 0)),
    out_shape=jax.ShapeDtypeStruct((B, DIM), jnp.float32),
)


def kernel(idx, vals, weight):
    idx_flat = idx.astype(jnp.int32).reshape(-1)
    vals_flat = vals.reshape(-1)
    h = _hist_kernel(idx_flat, vals_flat)
    return _matmul(h, weight)
